# Initial kernel scaffold; baseline (speedup 1.0000x reference)
import jax
import jax.numpy as jnp
from jax.experimental import pallas as pl


def kernel(x, edge_index, W1, att_src1, att_dst1, b1, W2, att_src2, att_dst2, b2):
    raise NotImplementedError("write your pallas kernel here")


# speedup vs baseline: 44.4268x; 44.4268x over previous
import functools

import jax
import jax.numpy as jnp
from jax import lax
from jax.experimental import pallas as pl
from jax.experimental.pallas import tpu as pltpu
from jax.experimental.pallas import tpu_sc as plsc

N = 10000
D = 128
NC, NS, LANES = 2, 16, 16
NW = NC * NS
CH = 128
BN = 1000
N_PAD = 10240
ROWS_PER_TILE = N_PAD // NS


def _leaky(v):
    return jnp.maximum(v, v * 0.2)


def _prep_block(xp, atu_ref, t_ref, u_ref, mt_ref, mu_ref, step):
    tu = jnp.dot(xp, atu_ref[...], preferred_element_type=jnp.float32)
    t = tu[:, :16]
    u = tu[:, 16:]
    t_ref[...] = t
    u_ref[...] = u
    bt = jnp.max(t, axis=0, keepdims=True)
    bu = jnp.max(u, axis=0, keepdims=True)

    @pl.when(step == 0)
    def _():
        mt_ref[...] = bt
        mu_ref[...] = bu

    @pl.when(step > 0)
    def _():
        mt_ref[...] = jnp.maximum(mt_ref[...], bt)
        mu_ref[...] = jnp.maximum(mu_ref[...], bu)


def _prep_body(x_ref, w_ref, atu_ref, xp_ref, t_ref, u_ref, mt_ref, mu_ref):
    i = pl.program_id(0)
    xp = jnp.dot(x_ref[...], w_ref[...], preferred_element_type=jnp.float32)
    xp_ref[...] = xp
    _prep_block(xp, atu_ref, t_ref, u_ref, mt_ref, mu_ref, i)


def _finalize(msg_ref, den_ref, xp_ref, t_ref, u_ref, mt_ref, mu_ref, b_ref,
              erep_ref):
    m16 = _leaky(mt_ref[...] + mu_ref[...])
    p_self = jnp.exp(_leaky(t_ref[...][:, :8] + u_ref[...][:, :8])
                     - m16[:, :8])
    msum = msg_ref[0] + msg_ref[1]
    dsum = den_ref[0][:, :8] + den_ref[1][:, :8] + p_self
    erep = erep_ref[...]
    pexp = jnp.dot(p_self, erep, preferred_element_type=jnp.float32)
    dexp = jnp.dot(dsum, erep, preferred_element_type=jnp.float32)
    return (msum + xp_ref[...] * pexp) / (dexp + 1e-16) + b_ref[...]


def _mid_body(msg_ref, den_ref, xp_ref, t_ref, u_ref, mt_ref, mu_ref, b_ref,
              erep_ref, w2_ref, atu2_ref,
              xp2_ref, t2_ref, u2_ref, mt2_ref, mu2_ref):
    i = pl.program_id(0)
    h = _finalize(msg_ref, den_ref, xp_ref, t_ref, u_ref, mt_ref, mu_ref,
                  b_ref, erep_ref)
    xp2 = jnp.dot(h, w2_ref[...], preferred_element_type=jnp.float32)
    xp2_ref[...] = xp2
    _prep_block(xp2, atu2_ref, t2_ref, u2_ref, mt2_ref, mu2_ref, i)


def _fin_body(msg_ref, den_ref, xp_ref, t_ref, u_ref, mt_ref, mu_ref, b_ref,
              erep_ref, out_ref):
    out_ref[...] = _finalize(msg_ref, den_ref, xp_ref, t_ref, u_ref, mt_ref,
                             mu_ref, b_ref, erep_ref)


def _row_spec(w):
    return pl.BlockSpec((BN, w), lambda i: (i, 0))


def _full_spec(shape):
    nd = len(shape)
    return pl.BlockSpec(shape, lambda i, _n=nd: (0,) * _n)


_GRID = N // BN

_PREP_CALL = pl.pallas_call(
    _prep_body,
    grid=(_GRID,),
    in_specs=[_row_spec(128), _full_spec((128, 128)), _full_spec((128, 32))],
    out_specs=[_row_spec(128), _row_spec(16), _row_spec(16),
               _full_spec((1, 16)), _full_spec((1, 16))],
    out_shape=[jax.ShapeDtypeStruct((N, 128), jnp.float32),
               jax.ShapeDtypeStruct((N, 16), jnp.float32),
               jax.ShapeDtypeStruct((N, 16), jnp.float32),
               jax.ShapeDtypeStruct((1, 16), jnp.float32),
               jax.ShapeDtypeStruct((1, 16), jnp.float32)],
)

_ACC_SPECS = [pl.BlockSpec((2, BN, 128), lambda i: (0, i, 0)),
              pl.BlockSpec((2, BN, 16), lambda i: (0, i, 0)),
              _row_spec(128), _row_spec(16), _row_spec(16),
              _full_spec((1, 16)), _full_spec((1, 16)),
              _full_spec((1, 128)), _full_spec((8, 128))]

_MID_CALL = pl.pallas_call(
    _mid_body,
    grid=(_GRID,),
    in_specs=_ACC_SPECS + [_full_spec((128, 128)), _full_spec((128, 32))],
    out_specs=[_row_spec(128), _row_spec(16), _row_spec(16),
               _full_spec((1, 16)), _full_spec((1, 16))],
    out_shape=[jax.ShapeDtypeStruct((N, 128), jnp.float32),
               jax.ShapeDtypeStruct((N, 16), jnp.float32),
               jax.ShapeDtypeStruct((N, 16), jnp.float32),
               jax.ShapeDtypeStruct((1, 16), jnp.float32),
               jax.ShapeDtypeStruct((1, 16), jnp.float32)],
)

_FIN_CALL = pl.pallas_call(
    _fin_body,
    grid=(_GRID,),
    in_specs=_ACC_SPECS,
    out_specs=_row_spec(128),
    out_shape=jax.ShapeDtypeStruct((N, 128), jnp.float32),
)


def _sc_edges_body(heads, src_h, dst_h, xp_h, t_h, u_h, mt_h, mu_h,
                   zmsg_h, zden_h, msg_out, den_out,
                   msg_s, den_s,
                   src_v, dst_v, deff_v, pen_v, ts_v, ud_v, rows_v, p_v,
                   mt_v, mu_v, gsem):
    cid = lax.axis_index("c")
    sid = lax.axis_index("s")
    wid = sid * NC + cid

    row0 = sid * ROWS_PER_TILE
    pltpu.sync_copy(zmsg_h, msg_s.at[pl.ds(row0, ROWS_PER_TILE), :])
    pltpu.sync_copy(zden_h, den_s.at[pl.ds(row0, ROWS_PER_TILE), :])

    pltpu.sync_copy(mt_h, mt_v)
    pltpu.sync_copy(mu_h, mu_v)
    m16 = _leaky(mt_v[...] + mu_v[...])

    plsc.subcore_barrier()

    n_chunks = src_h.shape[0] // CH
    cnt = (n_chunks // NW) + jnp.where(wid < (n_chunks % NW), 1, 0)

    def chunk_body(i, _):
        base = (wid + i * NW) * CH
        pltpu.sync_copy(src_h.at[pl.ds(base, CH)], src_v)
        pltpu.sync_copy(dst_h.at[pl.ds(base, CH)], dst_v)

        c1 = pltpu.async_copy(t_h.at[src_v], ts_v, gsem)
        c2 = pltpu.async_copy(u_h.at[dst_v], ud_v, gsem)
        c3 = pltpu.async_copy(xp_h.at[src_v], rows_v, gsem)

        for g in range(CH // LANES):
            s16 = src_v[pl.ds(g * LANES, LANES)]
            d16 = dst_v[pl.ds(g * LANES, LANES)]
            ok = s16 != d16
            deff_v[pl.ds(g * LANES, LANES)] = jnp.where(ok, d16, 0)
            pen_v[pl.ds(g * LANES, LANES)] = jnp.where(ok, 0.0, 1e30)

        c1.wait()
        c2.wait()
        c3.wait()

        def group_body(g, _):
            pen16 = pen_v[pl.ds(g * LANES, LANES)]
            for j in range(LANES):
                e = g * LANES + j
                al = ts_v[e, :] + ud_v[e, :]
                al = jnp.maximum(al, al * 0.2)
                p = jnp.exp(al - m16 - pen16[j])
                p_v[e, :] = p
                if heads == 8:
                    for h in range(8):
                        rows_v[e, pl.ds(h * LANES, LANES)] = (
                            rows_v[e, pl.ds(h * LANES, LANES)] * p[h])
                else:
                    f = p[0]
                    for h in range(8):
                        rows_v[e, pl.ds(h * LANES, LANES)] = (
                            rows_v[e, pl.ds(h * LANES, LANES)] * f)
            return 0

        lax.fori_loop(0, CH // LANES, group_body, 0)

        pltpu.sync_copy(rows_v, msg_s.at[deff_v], add=True)
        pltpu.sync_copy(p_v, den_s.at[deff_v], add=True)
        return 0

    lax.fori_loop(0, cnt, chunk_body, 0)

    plsc.subcore_barrier()

    pltpu.sync_copy(msg_s.at[pl.ds(row0, ROWS_PER_TILE), :],
                    msg_out.at[cid, pl.ds(row0, ROWS_PER_TILE), :])
    pltpu.sync_copy(den_s.at[pl.ds(row0, ROWS_PER_TILE), :],
                    den_out.at[cid, pl.ds(row0, ROWS_PER_TILE), :])


def _make_sc_call(heads):
    mesh = plsc.VectorSubcoreMesh(core_axis_name="c", subcore_axis_name="s",
                                  num_cores=NC, num_subcores=NS)
    return pl.kernel(
        functools.partial(_sc_edges_body, heads),
        out_type=[jax.ShapeDtypeStruct((NC, N_PAD, 128), jnp.float32),
                  jax.ShapeDtypeStruct((NC, N_PAD, 16), jnp.float32)],
        mesh=mesh,
        scratch_types=[
            pltpu.VMEM_SHARED((N_PAD, 128), jnp.float32),
            pltpu.VMEM_SHARED((N_PAD, 16), jnp.float32),
            pltpu.VMEM((CH,), jnp.int32),
            pltpu.VMEM((CH,), jnp.int32),
            pltpu.VMEM((CH,), jnp.int32),
            pltpu.VMEM((CH,), jnp.float32),
            pltpu.VMEM((CH, 16), jnp.float32),
            pltpu.VMEM((CH, 16), jnp.float32),
            pltpu.VMEM((CH, 128), jnp.float32),
            pltpu.VMEM((CH, 16), jnp.float32),
            pltpu.VMEM((16,), jnp.float32),
            pltpu.VMEM((16,), jnp.float32),
            pltpu.SemaphoreType.DMA,
        ],
        compiler_params=pltpu.CompilerParams(use_tc_tiling_on_sc=False),
    )


@functools.cache
def _sc_call(heads):
    return _make_sc_call(heads)


def _expand_block_diag(att):
    h, c = att.shape[1], att.shape[2]
    a = att[0]
    if h == 8:
        m = a[:, :, None] * jnp.eye(8, dtype=a.dtype)[:, None, :]
        return m.reshape(128, 8)
    return jnp.tile(a.reshape(128, 1), (1, 8))


def _atu(att_src, att_dst):
    s = _expand_block_diag(att_src)
    d = _expand_block_diag(att_dst)
    return jnp.concatenate([s, d, d, s], axis=1)


def kernel(x, edge_index, W1, att_src1, att_dst1, b1,
           W2, att_src2, att_dst2, b2):
    src = edge_index[0]
    dst = edge_index[1]
    atu1 = _atu(att_src1, att_dst1)
    atu2 = _atu(att_src2, att_dst2)
    erep = jnp.repeat(jnp.eye(8, dtype=jnp.float32), 16, axis=1)
    zmsg = jnp.zeros((ROWS_PER_TILE, 128), jnp.float32)
    zden = jnp.zeros((ROWS_PER_TILE, 16), jnp.float32)
    b1r = b1.reshape(1, 128)
    b2r = b2.reshape(1, 128)

    xp1, t1, u1, mt1, mu1 = _PREP_CALL(x, W1, atu1)
    msg1, den1 = _sc_call(8)(src, dst, xp1, t1, u1,
                             mt1.reshape(16), mu1.reshape(16), zmsg, zden)
    msg1, den1 = msg1[:, :N], den1[:, :N]
    xp2, t2, u2, mt2, mu2 = _MID_CALL(msg1, den1, xp1, t1, u1, mt1, mu1,
                                      b1r, erep, W2, atu2)
    msg2, den2 = _sc_call(1)(src, dst, xp2, t2, u2,
                             mt2.reshape(16), mu2.reshape(16), zmsg, zden)
    msg2, den2 = msg2[:, :N], den2[:, :N]
    return _FIN_CALL(msg2, den2, xp2, t2, u2, mt2, mu2, b2r, erep)


# speedup vs baseline: 65.5355x; 1.4751x over previous
import functools

import jax
import jax.numpy as jnp
from jax import lax
from jax.experimental import pallas as pl
from jax.experimental.pallas import tpu as pltpu
from jax.experimental.pallas import tpu_sc as plsc

N = 10000
D = 128
NC, NS, LANES = 2, 16, 16
NW = NC * NS
CH = 96
BN = 1000
N_PAD = 10112
ROWS_PER_TILE = N_PAD // NS


def _leaky(v):
    return jnp.maximum(v, v * 0.2)


def _prep_block(xp, atu_ref, t_ref, u_ref, mt_ref, mu_ref, step):
    tu = jnp.dot(xp, atu_ref[...], preferred_element_type=jnp.float32)
    t = tu[:, :16]
    u = tu[:, 16:]
    t_ref[...] = t
    u_ref[...] = u
    bt = jnp.max(t, axis=0, keepdims=True)
    bu = jnp.max(u, axis=0, keepdims=True)

    @pl.when(step == 0)
    def _():
        mt_ref[...] = bt
        mu_ref[...] = bu

    @pl.when(step > 0)
    def _():
        mt_ref[...] = jnp.maximum(mt_ref[...], bt)
        mu_ref[...] = jnp.maximum(mu_ref[...], bu)


def _prep_body(x_ref, w_ref, atu_ref, xp_ref, t_ref, u_ref, mt_ref, mu_ref):
    i = pl.program_id(0)
    xp = jnp.dot(x_ref[...], w_ref[...], preferred_element_type=jnp.float32)
    xp_ref[...] = xp
    _prep_block(xp, atu_ref, t_ref, u_ref, mt_ref, mu_ref, i)


def _finalize(msg_ref, den_ref, xp_ref, t_ref, u_ref, mt_ref, mu_ref, b_ref,
              erep_ref):
    m16 = _leaky(mt_ref[...] + mu_ref[...])
    p_self = jnp.exp(_leaky(t_ref[...][:, :8] + u_ref[...][:, :8])
                     - m16[:, :8])
    msum = msg_ref[0] + msg_ref[1]
    dsum = den_ref[0][:, :8] + den_ref[1][:, :8] + p_self
    erep = erep_ref[...]
    pexp = jnp.dot(p_self, erep, preferred_element_type=jnp.float32)
    dexp = jnp.dot(dsum, erep, preferred_element_type=jnp.float32)
    return (msum + xp_ref[...] * pexp) / (dexp + 1e-16) + b_ref[...]


def _mid_body(msg_ref, den_ref, xp_ref, t_ref, u_ref, mt_ref, mu_ref, b_ref,
              erep_ref, w2_ref, atu2_ref,
              xp2_ref, t2_ref, u2_ref, mt2_ref, mu2_ref):
    i = pl.program_id(0)
    h = _finalize(msg_ref, den_ref, xp_ref, t_ref, u_ref, mt_ref, mu_ref,
                  b_ref, erep_ref)
    xp2 = jnp.dot(h, w2_ref[...], preferred_element_type=jnp.float32)
    xp2_ref[...] = xp2
    _prep_block(xp2, atu2_ref, t2_ref, u2_ref, mt2_ref, mu2_ref, i)


def _fin_body(msg_ref, den_ref, xp_ref, t_ref, u_ref, mt_ref, mu_ref, b_ref,
              erep_ref, out_ref):
    out_ref[...] = _finalize(msg_ref, den_ref, xp_ref, t_ref, u_ref, mt_ref,
                             mu_ref, b_ref, erep_ref)


def _row_spec(w):
    return pl.BlockSpec((BN, w), lambda i: (i, 0))


def _full_spec(shape):
    nd = len(shape)
    return pl.BlockSpec(shape, lambda i, _n=nd: (0,) * _n)


_GRID = N // BN

_PREP_CALL = pl.pallas_call(
    _prep_body,
    grid=(_GRID,),
    in_specs=[_row_spec(128), _full_spec((128, 128)), _full_spec((128, 32))],
    out_specs=[_row_spec(128), _row_spec(16), _row_spec(16),
               _full_spec((1, 16)), _full_spec((1, 16))],
    out_shape=[jax.ShapeDtypeStruct((N, 128), jnp.float32),
               jax.ShapeDtypeStruct((N, 16), jnp.float32),
               jax.ShapeDtypeStruct((N, 16), jnp.float32),
               jax.ShapeDtypeStruct((1, 16), jnp.float32),
               jax.ShapeDtypeStruct((1, 16), jnp.float32)],
)

_ACC_SPECS = [pl.BlockSpec((2, BN, 128), lambda i: (0, i, 0)),
              pl.BlockSpec((2, BN, 16), lambda i: (0, i, 0)),
              _row_spec(128), _row_spec(16), _row_spec(16),
              _full_spec((1, 16)), _full_spec((1, 16)),
              _full_spec((1, 128)), _full_spec((8, 128))]

_MID_CALL = pl.pallas_call(
    _mid_body,
    grid=(_GRID,),
    in_specs=_ACC_SPECS + [_full_spec((128, 128)), _full_spec((128, 32))],
    out_specs=[_row_spec(128), _row_spec(16), _row_spec(16),
               _full_spec((1, 16)), _full_spec((1, 16))],
    out_shape=[jax.ShapeDtypeStruct((N, 128), jnp.float32),
               jax.ShapeDtypeStruct((N, 16), jnp.float32),
               jax.ShapeDtypeStruct((N, 16), jnp.float32),
               jax.ShapeDtypeStruct((1, 16), jnp.float32),
               jax.ShapeDtypeStruct((1, 16), jnp.float32)],
)

_FIN_CALL = pl.pallas_call(
    _fin_body,
    grid=(_GRID,),
    in_specs=_ACC_SPECS,
    out_specs=_row_spec(128),
    out_shape=jax.ShapeDtypeStruct((N, 128), jnp.float32),
)


E_EDGES = 320000
E_PAD = ((E_EDGES + CH - 1) // CH) * CH


def _sc_edges_body(heads, src_h, dst_h, xp_h, t_h, u_h, mt_h, mu_h,
                   zmsg_h, zden_h, msg_out, den_out,
                   msg_s, den_s,
                   p_v,
                   src0, dst0, deff0, pen0, ts0, ud0, rows0,
                   src1, dst1, deff1, pen1, ts1, ud1, rows1,
                   mt_v, mu_v, sem0, sem1, isem):
    cid = lax.axis_index("c")
    sid = lax.axis_index("s")
    wid = sid * NC + cid

    row0 = sid * ROWS_PER_TILE
    pltpu.sync_copy(zmsg_h, msg_s.at[pl.ds(row0, ROWS_PER_TILE), :])
    pltpu.sync_copy(zden_h, den_s.at[pl.ds(row0, ROWS_PER_TILE), :])

    pltpu.sync_copy(mt_h, mt_v)
    pltpu.sync_copy(mu_h, mu_v)
    m16 = _leaky(mt_v[...] + mu_v[...])

    n_chunks = src_h.shape[0] // CH
    base_cnt = n_chunks // NW
    rem = n_chunks % NW
    cnt = base_cnt + jnp.where(wid < rem, 1, 0)

    plsc.subcore_barrier()

    bufs = [(src0, dst0, deff0, pen0, ts0, ud0, rows0, sem0),
            (src1, dst1, deff1, pen1, ts1, ud1, rows1, sem1)]

    def edge_base(c):
        return (wid + c * NW) * CH

    def idx_load(c, b, sync):
        (src_v, dst_v) = b[0], b[1]
        base = edge_base(c)
        if sync:
            pltpu.sync_copy(src_h.at[pl.ds(base, CH)], src_v)
            pltpu.sync_copy(dst_h.at[pl.ds(base, CH)], dst_v)
        else:
            pltpu.async_copy(src_h.at[pl.ds(base, CH)], src_v, isem)
            pltpu.async_copy(dst_h.at[pl.ds(base, CH)], dst_v, isem)

    def idx_wait(b):
        (src_v, dst_v) = b[0], b[1]
        pltpu.make_async_copy(src_h.at[pl.ds(0, CH)], src_v, isem).wait()
        pltpu.make_async_copy(dst_h.at[pl.ds(0, CH)], dst_v, isem).wait()

    def fire_and_mask(b):
        (src_v, dst_v, deff_v, pen_v, ts_v, ud_v, rows_v, sem) = b
        pltpu.async_copy(t_h.at[src_v], ts_v, sem)
        pltpu.async_copy(u_h.at[dst_v], ud_v, sem)
        pltpu.async_copy(xp_h.at[src_v], rows_v, sem)
        for g in range(CH // LANES):
            s16 = src_v[pl.ds(g * LANES, LANES)]
            d16 = dst_v[pl.ds(g * LANES, LANES)]
            ok = s16 != d16
            deff_v[pl.ds(g * LANES, LANES)] = jnp.where(ok, d16, 0)
            pen_v[pl.ds(g * LANES, LANES)] = jnp.where(ok, 0.0, 1e30)

    def gather_wait(b):
        (_, _, _, _, ts_v, ud_v, rows_v, sem) = b
        pltpu.make_async_copy(t_h.at[pl.ds(0, CH), :], ts_v, sem).wait()
        pltpu.make_async_copy(u_h.at[pl.ds(0, CH), :], ud_v, sem).wait()
        pltpu.make_async_copy(xp_h.at[pl.ds(0, CH), :], rows_v, sem).wait()

    def compute(b):
        (src_v, dst_v, deff_v, pen_v, ts_v, ud_v, rows_v, sem) = b

        def group_body(g, _):
            pen16 = pen_v[pl.ds(g * LANES, LANES)]
            for j in range(LANES):
                e = g * LANES + j
                al = ts_v[e, :] + ud_v[e, :]
                al = jnp.maximum(al, al * 0.2)
                p = jnp.exp(al - m16 - pen16[j])
                p_v[e, :] = p
                if heads == 8:
                    for h in range(8):
                        rows_v[e, pl.ds(h * LANES, LANES)] = (
                            rows_v[e, pl.ds(h * LANES, LANES)] * p[h])
                else:
                    f = p[0]
                    for h in range(8):
                        rows_v[e, pl.ds(h * LANES, LANES)] = (
                            rows_v[e, pl.ds(h * LANES, LANES)] * f)
            return 0

        lax.fori_loop(0, CH // LANES, group_body, 0)

        pltpu.sync_copy(rows_v, msg_s.at[deff_v], add=True)
        pltpu.sync_copy(p_v, den_s.at[deff_v], add=True)

    idx_load(0, bufs[0], sync=True)
    fire_and_mask(bufs[0])

    @pl.when(cnt > 1)
    def _():
        idx_load(1, bufs[1], sync=False)

    def chunk_body(i, _):
        even = (i % 2) == 0
        more = (i + 1) < cnt
        more2 = (i + 2) < cnt

        def stage(cur, nxt):
            @pl.when(more)
            def _():
                idx_wait(nxt)
                fire_and_mask(nxt)

            gather_wait(cur)

            @pl.when(more2)
            def _():
                idx_load(i + 2, cur, sync=False)

            compute(cur)

        @pl.when(even)
        def _():
            stage(bufs[0], bufs[1])

        @pl.when(jnp.logical_not(even))
        def _():
            stage(bufs[1], bufs[0])

        return 0

    lax.fori_loop(0, cnt, chunk_body, 0)

    plsc.subcore_barrier()

    pltpu.sync_copy(msg_s.at[pl.ds(row0, ROWS_PER_TILE), :],
                    msg_out.at[cid, pl.ds(row0, ROWS_PER_TILE), :])
    pltpu.sync_copy(den_s.at[pl.ds(row0, ROWS_PER_TILE), :],
                    den_out.at[cid, pl.ds(row0, ROWS_PER_TILE), :])


def _make_sc_call(heads):
    mesh = plsc.VectorSubcoreMesh(core_axis_name="c", subcore_axis_name="s",
                                  num_cores=NC, num_subcores=NS)
    return pl.kernel(
        functools.partial(_sc_edges_body, heads),
        out_type=[jax.ShapeDtypeStruct((NC, N_PAD, 128), jnp.float32),
                  jax.ShapeDtypeStruct((NC, N_PAD, 16), jnp.float32)],
        mesh=mesh,
        scratch_types=[
            pltpu.VMEM_SHARED((N_PAD, 128), jnp.float32),
            pltpu.VMEM_SHARED((N_PAD, 16), jnp.float32),
            pltpu.VMEM((CH, 16), jnp.float32),
        ] + 2 * [
            pltpu.VMEM((CH,), jnp.int32),
            pltpu.VMEM((CH,), jnp.int32),
            pltpu.VMEM((CH,), jnp.int32),
            pltpu.VMEM((CH,), jnp.float32),
            pltpu.VMEM((CH, 16), jnp.float32),
            pltpu.VMEM((CH, 16), jnp.float32),
            pltpu.VMEM((CH, 128), jnp.float32),
        ] + [
            pltpu.VMEM((16,), jnp.float32),
            pltpu.VMEM((16,), jnp.float32),
            pltpu.SemaphoreType.DMA,
            pltpu.SemaphoreType.DMA,
            pltpu.SemaphoreType.DMA,
        ],
        compiler_params=pltpu.CompilerParams(use_tc_tiling_on_sc=False),
    )


@functools.cache
def _sc_call(heads):
    return _make_sc_call(heads)


def _expand_block_diag(att):
    h, c = att.shape[1], att.shape[2]
    a = att[0]
    if h == 8:
        m = a[:, :, None] * jnp.eye(8, dtype=a.dtype)[:, None, :]
        return m.reshape(128, 8)
    return jnp.tile(a.reshape(128, 1), (1, 8))


def _atu(att_src, att_dst):
    s = _expand_block_diag(att_src)
    d = _expand_block_diag(att_dst)
    return jnp.concatenate([s, d, d, s], axis=1)


def kernel(x, edge_index, W1, att_src1, att_dst1, b1,
           W2, att_src2, att_dst2, b2):
    pad = jnp.zeros((E_PAD - E_EDGES,), jnp.int32)
    src = jnp.concatenate([edge_index[0], pad])
    dst = jnp.concatenate([edge_index[1], pad])
    atu1 = _atu(att_src1, att_dst1)
    atu2 = _atu(att_src2, att_dst2)
    erep = jnp.repeat(jnp.eye(8, dtype=jnp.float32), 16, axis=1)
    zmsg = jnp.zeros((ROWS_PER_TILE, 128), jnp.float32)
    zden = jnp.zeros((ROWS_PER_TILE, 16), jnp.float32)
    b1r = b1.reshape(1, 128)
    b2r = b2.reshape(1, 128)

    xp1, t1, u1, mt1, mu1 = _PREP_CALL(x, W1, atu1)
    msg1, den1 = _sc_call(8)(src, dst, xp1, t1, u1,
                             mt1.reshape(16), mu1.reshape(16), zmsg, zden)
    msg1, den1 = msg1[:, :N], den1[:, :N]
    xp2, t2, u2, mt2, mu2 = _MID_CALL(msg1, den1, xp1, t1, u1, mt1, mu1,
                                      b1r, erep, W2, atu2)
    msg2, den2 = _sc_call(1)(src, dst, xp2, t2, u2,
                             mt2.reshape(16), mu2.reshape(16), zmsg, zden)
    msg2, den2 = msg2[:, :N], den2[:, :N]
    return _FIN_CALL(msg2, den2, xp2, t2, u2, mt2, mu2, b2r, erep)


# speedup vs baseline: 66.5600x; 1.0156x over previous
import functools

import jax
import jax.numpy as jnp
from jax import lax
from jax.experimental import pallas as pl
from jax.experimental.pallas import tpu as pltpu
from jax.experimental.pallas import tpu_sc as plsc

N = 10000
D = 128
NC, NS, LANES = 2, 16, 16
NW = NC * NS
CH = 96
BN = 1000
N_PAD = 10112
ROWS_PER_TILE = N_PAD // NS


def _leaky(v):
    return jnp.maximum(v, v * 0.2)


def _prep_block(xp, atu_ref, t_ref, u_ref, mt_ref, mu_ref, step):
    tu = jnp.dot(xp, atu_ref[...], preferred_element_type=jnp.float32)
    t = tu[:, :16]
    u = tu[:, 16:]
    t_ref[...] = t
    u_ref[...] = u
    bt = jnp.max(t, axis=0, keepdims=True)
    bu = jnp.max(u, axis=0, keepdims=True)

    @pl.when(step == 0)
    def _():
        mt_ref[...] = bt
        mu_ref[...] = bu

    @pl.when(step > 0)
    def _():
        mt_ref[...] = jnp.maximum(mt_ref[...], bt)
        mu_ref[...] = jnp.maximum(mu_ref[...], bu)


def _prep_body(x_ref, w_ref, atu_ref, xp_ref, t_ref, u_ref, mt_ref, mu_ref):
    i = pl.program_id(0)
    xp = jnp.dot(x_ref[...], w_ref[...], preferred_element_type=jnp.float32)
    xp_ref[...] = xp
    _prep_block(xp, atu_ref, t_ref, u_ref, mt_ref, mu_ref, i)


def _finalize(msg_ref, den_ref, xp_ref, t_ref, u_ref, mt_ref, mu_ref, b_ref,
              erep_ref):
    m16 = _leaky(mt_ref[...] + mu_ref[...])
    p_self = jnp.exp(_leaky(t_ref[...][:, :8] + u_ref[...][:, :8])
                     - m16[:, :8])
    msum = msg_ref[0] + msg_ref[1]
    dsum = den_ref[0][:, :8] + den_ref[1][:, :8] + p_self
    erep = erep_ref[...]
    pexp = jnp.dot(p_self, erep, preferred_element_type=jnp.float32)
    dexp = jnp.dot(dsum, erep, preferred_element_type=jnp.float32)
    return (msum + xp_ref[...] * pexp) / (dexp + 1e-16) + b_ref[...]


def _mid_body(msg_ref, den_ref, xp_ref, t_ref, u_ref, mt_ref, mu_ref, b_ref,
              erep_ref, w2_ref, atu2_ref,
              xp2_ref, t2_ref, u2_ref, mt2_ref, mu2_ref):
    i = pl.program_id(0)
    h = _finalize(msg_ref, den_ref, xp_ref, t_ref, u_ref, mt_ref, mu_ref,
                  b_ref, erep_ref)
    xp2 = jnp.dot(h, w2_ref[...], preferred_element_type=jnp.float32)
    xp2_ref[...] = xp2
    _prep_block(xp2, atu2_ref, t2_ref, u2_ref, mt2_ref, mu2_ref, i)


def _fin_body(msg_ref, den_ref, xp_ref, t_ref, u_ref, mt_ref, mu_ref, b_ref,
              erep_ref, out_ref):
    out_ref[...] = _finalize(msg_ref, den_ref, xp_ref, t_ref, u_ref, mt_ref,
                             mu_ref, b_ref, erep_ref)


def _row_spec(w):
    return pl.BlockSpec((BN, w), lambda i: (i, 0))


def _full_spec(shape):
    nd = len(shape)
    return pl.BlockSpec(shape, lambda i, _n=nd: (0,) * _n)


_GRID = N // BN

_PREP_CALL = pl.pallas_call(
    _prep_body,
    grid=(_GRID,),
    in_specs=[_row_spec(128), _full_spec((128, 128)), _full_spec((128, 32))],
    out_specs=[_row_spec(128), _row_spec(16), _row_spec(16),
               _full_spec((1, 16)), _full_spec((1, 16))],
    out_shape=[jax.ShapeDtypeStruct((N, 128), jnp.float32),
               jax.ShapeDtypeStruct((N, 16), jnp.float32),
               jax.ShapeDtypeStruct((N, 16), jnp.float32),
               jax.ShapeDtypeStruct((1, 16), jnp.float32),
               jax.ShapeDtypeStruct((1, 16), jnp.float32)],
)

_ACC_SPECS = [pl.BlockSpec((2, BN, 128), lambda i: (0, i, 0)),
              pl.BlockSpec((2, BN, 16), lambda i: (0, i, 0)),
              _row_spec(128), _row_spec(16), _row_spec(16),
              _full_spec((1, 16)), _full_spec((1, 16)),
              _full_spec((1, 128)), _full_spec((8, 128))]

_MID_CALL = pl.pallas_call(
    _mid_body,
    grid=(_GRID,),
    in_specs=_ACC_SPECS + [_full_spec((128, 128)), _full_spec((128, 32))],
    out_specs=[_row_spec(128), _row_spec(16), _row_spec(16),
               _full_spec((1, 16)), _full_spec((1, 16))],
    out_shape=[jax.ShapeDtypeStruct((N, 128), jnp.float32),
               jax.ShapeDtypeStruct((N, 16), jnp.float32),
               jax.ShapeDtypeStruct((N, 16), jnp.float32),
               jax.ShapeDtypeStruct((1, 16), jnp.float32),
               jax.ShapeDtypeStruct((1, 16), jnp.float32)],
)

_FIN_CALL = pl.pallas_call(
    _fin_body,
    grid=(_GRID,),
    in_specs=_ACC_SPECS,
    out_specs=_row_spec(128),
    out_shape=jax.ShapeDtypeStruct((N, 128), jnp.float32),
)


E_EDGES = 320000
E_PAD = ((E_EDGES + CH - 1) // CH) * CH


def _sc_edges_body(heads, src_h, dst_h, xp_h, t_h, u_h, mt_h, mu_h,
                   zmsg_h, zden_h, msg_out, den_out,
                   msg_s, den_s,
                   src0, dst0, deff0, pen0, ts0, ud0, rows0, p0, sem0, ssem0,
                   src1, dst1, deff1, pen1, ts1, ud1, rows1, p1, sem1, ssem1,
                   mt_v, mu_v, isem):
    cid = lax.axis_index("c")
    sid = lax.axis_index("s")
    wid = sid * NC + cid

    row0 = sid * ROWS_PER_TILE
    pltpu.sync_copy(zmsg_h, msg_s.at[pl.ds(row0, ROWS_PER_TILE), :])
    pltpu.sync_copy(zden_h, den_s.at[pl.ds(row0, ROWS_PER_TILE), :])

    pltpu.sync_copy(mt_h, mt_v)
    pltpu.sync_copy(mu_h, mu_v)
    m16 = _leaky(mt_v[...] + mu_v[...])

    n_chunks = src_h.shape[0] // CH
    base_cnt = n_chunks // NW
    rem = n_chunks % NW
    cnt = base_cnt + jnp.where(wid < rem, 1, 0)

    plsc.subcore_barrier()

    bufs = [(src0, dst0, deff0, pen0, ts0, ud0, rows0, p0, sem0, ssem0),
            (src1, dst1, deff1, pen1, ts1, ud1, rows1, p1, sem1, ssem1)]

    def edge_base(c):
        return (wid + c * NW) * CH

    def idx_load(c, b, sync):
        (src_v, dst_v) = b[0], b[1]
        base = edge_base(c)
        if sync:
            pltpu.sync_copy(src_h.at[pl.ds(base, CH)], src_v)
            pltpu.sync_copy(dst_h.at[pl.ds(base, CH)], dst_v)
        else:
            pltpu.async_copy(src_h.at[pl.ds(base, CH)], src_v, isem)
            pltpu.async_copy(dst_h.at[pl.ds(base, CH)], dst_v, isem)

    def idx_wait(b):
        (src_v, dst_v) = b[0], b[1]
        pltpu.make_async_copy(src_h.at[pl.ds(0, CH)], src_v, isem).wait()
        pltpu.make_async_copy(dst_h.at[pl.ds(0, CH)], dst_v, isem).wait()

    def scatter_wait(b):
        (_, _, deff_v, _, _, _, rows_v, p_v, _, ssem) = b
        pltpu.make_async_copy(rows_v, msg_s.at[deff_v], ssem).wait()
        pltpu.make_async_copy(p_v, den_s.at[deff_v], ssem).wait()

    def fire_and_mask(b):
        (src_v, dst_v, deff_v, pen_v, ts_v, ud_v, rows_v, p_v, sem, ssem) = b
        pltpu.async_copy(t_h.at[src_v], ts_v, sem)
        pltpu.async_copy(u_h.at[dst_v], ud_v, sem)
        pltpu.async_copy(xp_h.at[src_v], rows_v, sem)
        for g in range(CH // LANES):
            s16 = src_v[pl.ds(g * LANES, LANES)]
            d16 = dst_v[pl.ds(g * LANES, LANES)]
            ok = s16 != d16
            deff_v[pl.ds(g * LANES, LANES)] = jnp.where(ok, d16, 0)
            pen_v[pl.ds(g * LANES, LANES)] = jnp.where(ok, 0.0, 1e30)

    def gather_wait(b):
        (_, _, _, _, ts_v, ud_v, rows_v, _, sem, _) = b
        pltpu.make_async_copy(t_h.at[pl.ds(0, CH), :], ts_v, sem).wait()
        pltpu.make_async_copy(u_h.at[pl.ds(0, CH), :], ud_v, sem).wait()
        pltpu.make_async_copy(xp_h.at[pl.ds(0, CH), :], rows_v, sem).wait()

    def compute(b):
        (src_v, dst_v, deff_v, pen_v, ts_v, ud_v, rows_v, p_v, sem, ssem) = b

        def group_body(g, _):
            pen16 = pen_v[pl.ds(g * LANES, LANES)]
            for j in range(LANES):
                e = g * LANES + j
                al = ts_v[e, :] + ud_v[e, :]
                al = jnp.maximum(al, al * 0.2)
                p = jnp.exp(al - m16 - pen16[j])
                p_v[e, :] = p
                if heads == 8:
                    for h in range(8):
                        rows_v[e, pl.ds(h * LANES, LANES)] = (
                            rows_v[e, pl.ds(h * LANES, LANES)] * p[h])
                else:
                    f = p[0]
                    for h in range(8):
                        rows_v[e, pl.ds(h * LANES, LANES)] = (
                            rows_v[e, pl.ds(h * LANES, LANES)] * f)
            return 0

        lax.fori_loop(0, CH // LANES, group_body, 0)

        pltpu.async_copy(rows_v, msg_s.at[deff_v], ssem, add=True)
        pltpu.async_copy(p_v, den_s.at[deff_v], ssem, add=True)

    idx_load(0, bufs[0], sync=True)
    fire_and_mask(bufs[0])

    @pl.when(cnt > 1)
    def _():
        idx_load(1, bufs[1], sync=False)

    def chunk_body(i, _):
        even = (i % 2) == 0
        more = (i + 1) < cnt
        more2 = (i + 2) < cnt

        def stage(cur, nxt):
            @pl.when(jnp.logical_and(more, i >= 1))
            def _():
                scatter_wait(nxt)

            @pl.when(more)
            def _():
                idx_wait(nxt)
                fire_and_mask(nxt)

            gather_wait(cur)

            @pl.when(more2)
            def _():
                idx_load(i + 2, cur, sync=False)

            compute(cur)

        @pl.when(even)
        def _():
            stage(bufs[0], bufs[1])

        @pl.when(jnp.logical_not(even))
        def _():
            stage(bufs[1], bufs[0])

        return 0

    lax.fori_loop(0, cnt, chunk_body, 0)

    @pl.when(cnt > 0)
    def _():
        scatter_wait(bufs[0])

    @pl.when(cnt > 1)
    def _():
        scatter_wait(bufs[1])

    plsc.subcore_barrier()

    pltpu.sync_copy(msg_s.at[pl.ds(row0, ROWS_PER_TILE), :],
                    msg_out.at[cid, pl.ds(row0, ROWS_PER_TILE), :])
    pltpu.sync_copy(den_s.at[pl.ds(row0, ROWS_PER_TILE), :],
                    den_out.at[cid, pl.ds(row0, ROWS_PER_TILE), :])


def _make_sc_call(heads):
    mesh = plsc.VectorSubcoreMesh(core_axis_name="c", subcore_axis_name="s",
                                  num_cores=NC, num_subcores=NS)
    return pl.kernel(
        functools.partial(_sc_edges_body, heads),
        out_type=[jax.ShapeDtypeStruct((NC, N_PAD, 128), jnp.float32),
                  jax.ShapeDtypeStruct((NC, N_PAD, 16), jnp.float32)],
        mesh=mesh,
        scratch_types=[
            pltpu.VMEM_SHARED((N_PAD, 128), jnp.float32),
            pltpu.VMEM_SHARED((N_PAD, 16), jnp.float32),
        ] + 2 * [
            pltpu.VMEM((CH,), jnp.int32),
            pltpu.VMEM((CH,), jnp.int32),
            pltpu.VMEM((CH,), jnp.int32),
            pltpu.VMEM((CH,), jnp.float32),
            pltpu.VMEM((CH, 16), jnp.float32),
            pltpu.VMEM((CH, 16), jnp.float32),
            pltpu.VMEM((CH, 128), jnp.float32),
            pltpu.VMEM((CH, 16), jnp.float32),
            pltpu.SemaphoreType.DMA,
            pltpu.SemaphoreType.DMA,
        ] + [
            pltpu.VMEM((16,), jnp.float32),
            pltpu.VMEM((16,), jnp.float32),
            pltpu.SemaphoreType.DMA,
        ],
        compiler_params=pltpu.CompilerParams(use_tc_tiling_on_sc=False),
    )


@functools.cache
def _sc_call(heads):
    return _make_sc_call(heads)


def _expand_block_diag(att):
    h, c = att.shape[1], att.shape[2]
    a = att[0]
    if h == 8:
        m = a[:, :, None] * jnp.eye(8, dtype=a.dtype)[:, None, :]
        return m.reshape(128, 8)
    return jnp.tile(a.reshape(128, 1), (1, 8))


def _atu(att_src, att_dst):
    s = _expand_block_diag(att_src)
    d = _expand_block_diag(att_dst)
    return jnp.concatenate([s, d, d, s], axis=1)


def kernel(x, edge_index, W1, att_src1, att_dst1, b1,
           W2, att_src2, att_dst2, b2):
    pad = jnp.zeros((E_PAD - E_EDGES,), jnp.int32)
    src = jnp.concatenate([edge_index[0], pad])
    dst = jnp.concatenate([edge_index[1], pad])
    atu1 = _atu(att_src1, att_dst1)
    atu2 = _atu(att_src2, att_dst2)
    erep = jnp.repeat(jnp.eye(8, dtype=jnp.float32), 16, axis=1)
    zmsg = jnp.zeros((ROWS_PER_TILE, 128), jnp.float32)
    zden = jnp.zeros((ROWS_PER_TILE, 16), jnp.float32)
    b1r = b1.reshape(1, 128)
    b2r = b2.reshape(1, 128)

    xp1, t1, u1, mt1, mu1 = _PREP_CALL(x, W1, atu1)
    msg1, den1 = _sc_call(8)(src, dst, xp1, t1, u1,
                             mt1.reshape(16), mu1.reshape(16), zmsg, zden)
    msg1, den1 = msg1[:, :N], den1[:, :N]
    xp2, t2, u2, mt2, mu2 = _MID_CALL(msg1, den1, xp1, t1, u1, mt1, mu1,
                                      b1r, erep, W2, atu2)
    msg2, den2 = _sc_call(1)(src, dst, xp2, t2, u2,
                             mt2.reshape(16), mu2.reshape(16), zmsg, zden)
    msg2, den2 = msg2[:, :N], den2[:, :N]
    return _FIN_CALL(msg2, den2, xp2, t2, u2, mt2, mu2, b2r, erep)
